# baseline (speedup 1.0000x reference)
import functools

import jax
import jax.numpy as jnp
from jax import lax
from jax.experimental import pallas as pl
from jax.experimental.pallas import tpu as pltpu
from jax.experimental.pallas import tpu_sc as plsc

_BATCH = 16384
_BITS = 128

_ROT_A = (13, 15, 26, 6)
_ROT_B = (17, 29, 16, 24)


def _threefry_bits(x1):
    ks = (jnp.uint32(0), jnp.uint32(42), jnp.uint32(0 ^ 42 ^ 0x1BD11BDA))

    def rotl(v, d):
        return (v << jnp.uint32(d)) | (v >> jnp.uint32(32 - d))

    x0 = x1
    x1 = x0 ^ rotl(x1, _ROT_A[0])
    for r in _ROT_A[1:]:
        x0 = x0 + x1
        x1 = rotl(x1, r)
        x1 = x0 ^ x1
    x0 = x0 + ks[1]
    x1 = x1 + (ks[2] + jnp.uint32(1))
    for i in range(1, 5):
        for r in (_ROT_A if i % 2 == 0 else _ROT_B):
            x0 = x0 + x1
            x1 = rotl(x1, r)
            x1 = x0 ^ x1
        x0 = x0 + ks[(i + 1) % 3]
        x1 = x1 + (ks[(i + 2) % 3] + jnp.uint32(i + 1))
    return x0 ^ x1


def _u_from_bits(bits):
    return jax.lax.bitcast_convert_type(
        (bits >> jnp.uint32(9)) | jnp.uint32(0x3F800000), jnp.float32) - 1.0


_TC_ROWS_PER_BLOCK = 1024


def _tc_body(x_ref, o_ref):
    p = pl.program_id(0)
    shape = (_TC_ROWS_PER_BLOCK, _BITS)
    base = (p * _TC_ROWS_PER_BLOCK * _BITS + 42).astype(jnp.uint32)
    x1 = base + (
        (jax.lax.broadcasted_iota(jnp.uint32, shape, 0) << jnp.uint32(7))
        + jax.lax.broadcasted_iota(jnp.uint32, shape, 1))
    u = _u_from_bits(_threefry_bits(x1))
    o_ref[...] = jnp.where(u < x_ref[...], 1.0, 0.0)


def _tc_sample(x2, rows):
    return pl.pallas_call(
        _tc_body,
        grid=(rows // _TC_ROWS_PER_BLOCK,),
        in_specs=[pl.BlockSpec((_TC_ROWS_PER_BLOCK, 1), lambda p: (p, 0))],
        out_specs=pl.BlockSpec((_TC_ROWS_PER_BLOCK, _BITS), lambda p: (p, 0)),
        out_shape=jax.ShapeDtypeStruct((_BATCH, _BITS), jnp.float32),
    )(x2)


_SC_WORKERS = 32


def _sc_body(row0, sc_rows, x_hbm, out0_hbm, out1_hbm, x_v, out_v):
    rows_per_c = sc_rows // 2
    rows_per_w = sc_rows // _SC_WORKERS
    cid = lax.axis_index("c")
    sid = lax.axis_index("s")
    cbase = cid * rows_per_c + sid * rows_per_w
    pltpu.sync_copy(x_hbm.at[pl.ds(row0 + cbase, rows_per_w)], x_v)
    lane = lax.iota(jnp.int32, 16)

    def group_body(g, carry):
        xs = x_v[pl.ds(g * 16, 16)]
        xb = [jnp.broadcast_to(xs[j], (16,)) for j in range(16)]
        gbase = (row0 + cbase + g * 16) * _BITS + 42

        def col_body(c, carry2):
            colbase = gbase + c * 16
            for j in range(16):
                ctr = jnp.full((16,), colbase + j * _BITS, jnp.int32) + lane
                u = _u_from_bits(_threefry_bits(ctr.astype(jnp.uint32)))
                out_v[g * 16 + j, pl.ds(c * 16, 16)] = \
                    jnp.where(u < xb[j], 1.0, 0.0)
            return carry2

        lax.fori_loop(0, _BITS // 16, col_body, 0)
        return carry

    lax.fori_loop(0, rows_per_w // 16, group_body, 0)

    @pl.when(cid == 0)
    def _():
        pltpu.sync_copy(out_v, out0_hbm.at[pl.ds(sid * rows_per_w, rows_per_w)])

    @pl.when(cid == 1)
    def _():
        pltpu.sync_copy(out_v, out1_hbm.at[pl.ds(sid * rows_per_w, rows_per_w)])


def _sc_sample(x, row0, sc_rows):
    rows_per_w = sc_rows // _SC_WORKERS
    mesh = plsc.VectorSubcoreMesh(core_axis_name="c", subcore_axis_name="s")
    f = pl.kernel(
        functools.partial(_sc_body, row0, sc_rows),
        out_type=[
            jax.ShapeDtypeStruct((sc_rows // 2, _BITS), jnp.float32),
            jax.ShapeDtypeStruct((sc_rows // 2, _BITS), jnp.float32),
        ],
        mesh=mesh,
        scratch_types=[
            pltpu.VMEM((rows_per_w,), jnp.float32),
            pltpu.VMEM((rows_per_w, _BITS), jnp.float32),
        ],
    )
    return f(x)


_SC_ROWS = 3072


def kernel(x):
    tc_rows = _BATCH - _SC_ROWS
    sc_out0, sc_out1 = _sc_sample(x, tc_rows, _SC_ROWS)
    out = _tc_sample(x.reshape(_BATCH, 1), tc_rows)
    out = jax.lax.dynamic_update_slice(out, sc_out0, (tc_rows, 0))
    out = jax.lax.dynamic_update_slice(out, sc_out1, (tc_rows + _SC_ROWS // 2, 0))
    return out


# speedup vs baseline: 1.1190x; 1.1190x over previous
import functools

import jax
import jax.numpy as jnp
from jax import lax
from jax.experimental import pallas as pl
from jax.experimental.pallas import tpu as pltpu
from jax.experimental.pallas import tpu_sc as plsc

_BATCH = 16384
_BITS = 128

_ROT_A = (13, 15, 26, 6)
_ROT_B = (17, 29, 16, 24)


def _threefry_bits(x1):
    ks = (jnp.uint32(0), jnp.uint32(42), jnp.uint32(0 ^ 42 ^ 0x1BD11BDA))

    def rotl(v, d):
        return (v << jnp.uint32(d)) | (v >> jnp.uint32(32 - d))

    x0 = x1
    x1 = x0 ^ rotl(x1, _ROT_A[0])
    for r in _ROT_A[1:]:
        x0 = x0 + x1
        x1 = rotl(x1, r)
        x1 = x0 ^ x1
    x0 = x0 + ks[1]
    x1 = x1 + (ks[2] + jnp.uint32(1))
    for i in range(1, 5):
        for r in (_ROT_A if i % 2 == 0 else _ROT_B):
            x0 = x0 + x1
            x1 = rotl(x1, r)
            x1 = x0 ^ x1
        x0 = x0 + ks[(i + 1) % 3]
        x1 = x1 + (ks[(i + 2) % 3] + jnp.uint32(i + 1))
    return x0 ^ x1


def _u_from_bits(bits):
    return jax.lax.bitcast_convert_type(
        (bits >> jnp.uint32(9)) | jnp.uint32(0x3F800000), jnp.float32) - 1.0


_TC_ROWS_PER_BLOCK = 1024


def _tc_body(x_ref, o_ref):
    p = pl.program_id(0)
    base0 = p * _TC_ROWS_PER_BLOCK * _BITS + 42
    shape = (_BITS, _BITS)
    tile_iota = (
        (jax.lax.broadcasted_iota(jnp.uint32, shape, 1) << jnp.uint32(7))
        + jax.lax.broadcasted_iota(jnp.uint32, shape, 0))
    for k in range(_TC_ROWS_PER_BLOCK // _BITS):
        base = jnp.uint32(base0 + k * _BITS * _BITS) + tile_iota
        u = _u_from_bits(_threefry_bits(base))
        xb = jnp.broadcast_to(x_ref[k:k + 1, :], shape)
        m = jnp.where(u < xb, 1.0, 0.0)
        o_ref[pl.ds(k * _BITS, _BITS), :] = m.T


def _tc_sample(x2, rows):
    return pl.pallas_call(
        _tc_body,
        grid=(rows // _TC_ROWS_PER_BLOCK,),
        in_specs=[pl.BlockSpec(
            (_TC_ROWS_PER_BLOCK // _BITS, _BITS), lambda p: (p, 0))],
        out_specs=pl.BlockSpec((_TC_ROWS_PER_BLOCK, _BITS), lambda p: (p, 0)),
        out_shape=jax.ShapeDtypeStruct((_BATCH, _BITS), jnp.float32),
    )(x2)


_SC_WORKERS = 32


def _sc_body(row0, sc_rows, x_hbm, out0_hbm, out1_hbm, x_v, out_v):
    rows_per_c = sc_rows // 2
    rows_per_w = sc_rows // _SC_WORKERS
    cid = lax.axis_index("c")
    sid = lax.axis_index("s")
    cbase = cid * rows_per_c + sid * rows_per_w
    pltpu.sync_copy(x_hbm.at[pl.ds(row0 + cbase, rows_per_w)], x_v)
    lane = lax.iota(jnp.int32, 16)

    def group_body(g, carry):
        xs = x_v[pl.ds(g * 16, 16)]
        xb = [jnp.broadcast_to(xs[j], (16,)) for j in range(16)]
        gbase = (row0 + cbase + g * 16) * _BITS + 42

        def col_body(c, carry2):
            colbase = gbase + c * 16
            for j in range(16):
                ctr = jnp.full((16,), colbase + j * _BITS, jnp.int32) + lane
                u = _u_from_bits(_threefry_bits(ctr.astype(jnp.uint32)))
                out_v[g * 16 + j, pl.ds(c * 16, 16)] = \
                    jnp.where(u < xb[j], 1.0, 0.0)
            return carry2

        lax.fori_loop(0, _BITS // 16, col_body, 0)
        return carry

    lax.fori_loop(0, rows_per_w // 16, group_body, 0)

    @pl.when(cid == 0)
    def _():
        pltpu.sync_copy(out_v, out0_hbm.at[pl.ds(sid * rows_per_w, rows_per_w)])

    @pl.when(cid == 1)
    def _():
        pltpu.sync_copy(out_v, out1_hbm.at[pl.ds(sid * rows_per_w, rows_per_w)])


def _sc_sample(x, row0, sc_rows):
    rows_per_w = sc_rows // _SC_WORKERS
    mesh = plsc.VectorSubcoreMesh(core_axis_name="c", subcore_axis_name="s")
    f = pl.kernel(
        functools.partial(_sc_body, row0, sc_rows),
        out_type=[
            jax.ShapeDtypeStruct((sc_rows // 2, _BITS), jnp.float32),
            jax.ShapeDtypeStruct((sc_rows // 2, _BITS), jnp.float32),
        ],
        mesh=mesh,
        scratch_types=[
            pltpu.VMEM((rows_per_w,), jnp.float32),
            pltpu.VMEM((rows_per_w, _BITS), jnp.float32),
        ],
    )
    return f(x)


_SC_ROWS = 3072


def kernel(x):
    tc_rows = _BATCH - _SC_ROWS
    sc_out0, sc_out1 = _sc_sample(x, tc_rows, _SC_ROWS)
    out = _tc_sample(x.reshape(_BATCH // _BITS, _BITS), tc_rows)
    out = jax.lax.dynamic_update_slice(out, sc_out0, (tc_rows, 0))
    out = jax.lax.dynamic_update_slice(out, sc_out1, (tc_rows + _SC_ROWS // 2, 0))
    return out


# speedup vs baseline: 1.5548x; 1.3895x over previous
import functools

import jax
import jax.numpy as jnp
from jax import lax
from jax.experimental import pallas as pl
from jax.experimental.pallas import tpu as pltpu
from jax.experimental.pallas import tpu_sc as plsc

_BATCH = 16384
_BITS = 128

_ROT_A = (13, 15, 26, 6)
_ROT_B = (17, 29, 16, 24)


def _threefry_bits(x1):
    ks = (jnp.uint32(0), jnp.uint32(42), jnp.uint32(0 ^ 42 ^ 0x1BD11BDA))

    def rotl(v, d):
        return (v << jnp.uint32(d)) | (v >> jnp.uint32(32 - d))

    x0 = x1
    x1 = x0 ^ rotl(x1, _ROT_A[0])
    for r in _ROT_A[1:]:
        x0 = x0 + x1
        x1 = rotl(x1, r)
        x1 = x0 ^ x1
    x0 = x0 + ks[1]
    x1 = x1 + (ks[2] + jnp.uint32(1))
    for i in range(1, 5):
        for r in (_ROT_A if i % 2 == 0 else _ROT_B):
            x0 = x0 + x1
            x1 = rotl(x1, r)
            x1 = x0 ^ x1
        x0 = x0 + ks[(i + 1) % 3]
        x1 = x1 + (ks[(i + 2) % 3] + jnp.uint32(i + 1))
    return x0 ^ x1


def _u_from_bits(bits):
    return jax.lax.bitcast_convert_type(
        (bits >> jnp.uint32(9)) | jnp.uint32(0x3F800000), jnp.float32) - 1.0


_TC_ROWS_PER_BLOCK = 1024


def _tc_body(x_ref, o_ref):
    p = pl.program_id(0)
    base0 = p * _TC_ROWS_PER_BLOCK * _BITS + 42
    shape = (_BITS, _BITS)
    tile_iota = (
        (jax.lax.broadcasted_iota(jnp.uint32, shape, 1) << jnp.uint32(7))
        + jax.lax.broadcasted_iota(jnp.uint32, shape, 0))
    for k in range(_TC_ROWS_PER_BLOCK // _BITS):
        base = jnp.uint32(base0 + k * _BITS * _BITS) + tile_iota
        u = _u_from_bits(_threefry_bits(base))
        xb = jnp.broadcast_to(x_ref[k:k + 1, :], shape)
        m = jnp.where(u < xb, 1.0, 0.0)
        o_ref[pl.ds(k * _BITS, _BITS), :] = m.T


def _tc_sample(x2, rows):
    return pl.pallas_call(
        _tc_body,
        grid=(rows // _TC_ROWS_PER_BLOCK,),
        in_specs=[pl.BlockSpec(
            (_TC_ROWS_PER_BLOCK // _BITS, _BITS), lambda p: (p, 0))],
        out_specs=pl.BlockSpec((_TC_ROWS_PER_BLOCK, _BITS), lambda p: (p, 0)),
        out_shape=jax.ShapeDtypeStruct((_BATCH, _BITS), jnp.float32),
    )(x2)


_SC_WORKERS = 32


def _sc_body(row0, sc_rows, x_hbm, out0_hbm, out1_hbm, x_v, out_v):
    rows_per_c = sc_rows // 2
    rows_per_w = sc_rows // _SC_WORKERS
    cid = lax.axis_index("c")
    sid = lax.axis_index("s")
    cbase = cid * rows_per_c + sid * rows_per_w
    pltpu.sync_copy(x_hbm.at[pl.ds(row0 + cbase, rows_per_w)], x_v)
    lane = lax.iota(jnp.int32, 16)
    lane128 = lane * _BITS

    def group_body(g, carry):
        lrow = g * 16 + lane
        xg = x_v[pl.ds(g * 16, 16)]
        ctr0 = jnp.full(
            (16,), (row0 + cbase + g * 16) * _BITS + 42, jnp.int32) + lane128

        xb = [jnp.broadcast_to(xg[j], (16,)) for j in range(16)]

        def col_body(c, carry2):
            cb = c * 16
            for j in range(16):
                ctr = ctr0 + (j * _BITS + cb)
                u = _u_from_bits(_threefry_bits(ctr.astype(jnp.uint32)))
                res = jnp.where(u < xb[j], 1.0, 0.0)
                fb = (g * 16 + j) * _BITS + cb
                out_v[pl.ds(fb, 16)] = res
            return carry2

        lax.fori_loop(0, _BITS // 16, col_body, 0)
        return carry

    lax.fori_loop(0, rows_per_w // 16, group_body, 0)

    nflat = rows_per_w * _BITS

    @pl.when(cid == 0)
    def _():
        pltpu.sync_copy(out_v, out0_hbm.at[pl.ds(sid * nflat, nflat)])

    @pl.when(cid == 1)
    def _():
        pltpu.sync_copy(out_v, out1_hbm.at[pl.ds(sid * nflat, nflat)])


def _sc_sample(x, row0, sc_rows):
    rows_per_w = sc_rows // _SC_WORKERS
    mesh = plsc.VectorSubcoreMesh(core_axis_name="c", subcore_axis_name="s")
    f = pl.kernel(
        functools.partial(_sc_body, row0, sc_rows),
        out_type=[
            jax.ShapeDtypeStruct((sc_rows // 2 * _BITS,), jnp.float32),
            jax.ShapeDtypeStruct((sc_rows // 2 * _BITS,), jnp.float32),
        ],
        mesh=mesh,
        scratch_types=[
            pltpu.VMEM((rows_per_w,), jnp.float32),
            pltpu.VMEM((rows_per_w * _BITS,), jnp.float32),
        ],
    )
    o0, o1 = f(x)
    return (o0.reshape(sc_rows // 2, _BITS), o1.reshape(sc_rows // 2, _BITS))


_SC_ROWS = 0


def kernel(x):
    tc_rows = _BATCH - _SC_ROWS
    if _SC_ROWS == 0:
        return _tc_sample(x.reshape(_BATCH // _BITS, _BITS), _BATCH)
    sc_out0, sc_out1 = _sc_sample(x, tc_rows, _SC_ROWS)
    out = _tc_sample(x.reshape(_BATCH // _BITS, _BITS), tc_rows)
    out = jax.lax.dynamic_update_slice(out, sc_out0, (tc_rows, 0))
    out = jax.lax.dynamic_update_slice(out, sc_out1, (tc_rows + _SC_ROWS // 2, 0))
    return out
